# baseline (speedup 1.0000x reference)
import functools

import jax
import jax.numpy as jnp
from jax import lax
from jax.experimental import pallas as pl
from jax.experimental.pallas import tpu as pltpu
from jax.experimental.pallas import tpu_sc as plsc

VOCAB = 1_000_000
HIDDEN = 64
B_TOTAL = 16384 * 200
NC, NS = 2, 16
NW = NC * NS
GROUP = 128
K = 8
CHUNK = K * GROUP
ROWS_PER_W = B_TOTAL // NW
STEPS = ROWS_PER_W // CHUNK


def _make_gather():
    mesh = plsc.VectorSubcoreMesh(core_axis_name="c", subcore_axis_name="s")

    @functools.partial(
        pl.kernel,
        mesh=mesh,
        out_type=jax.ShapeDtypeStruct((B_TOTAL, HIDDEN), jnp.float32),
        scratch_types=[
            pltpu.VMEM((K, GROUP), jnp.int32),
            pltpu.VMEM((CHUNK, HIDDEN), jnp.float32),
            pltpu.SemaphoreType.DMA,
        ],
        compiler_params=pltpu.CompilerParams(use_tc_tiling_on_sc=False),
    )
    def gather_kernel(table_hbm, idx_hbm, out_hbm, idx_v, rows_v, sem):
        wid = lax.axis_index("s") * NC + lax.axis_index("c")
        base_group = wid * (ROWS_PER_W // GROUP)

        def step(c, _):
            g0 = base_group + c * K
            pltpu.sync_copy(idx_hbm.at[pl.ds(g0, K)], idx_v)
            copies = [
                pltpu.async_copy(
                    table_hbm.at[idx_v.at[j]],
                    rows_v.at[pl.ds(j * GROUP, GROUP)],
                    sem,
                )
                for j in range(K)
            ]
            for cp in copies:
                cp.wait()
            pltpu.sync_copy(rows_v, out_hbm.at[pl.ds(g0 * GROUP, CHUNK)])
            return _

        lax.fori_loop(0, STEPS, step, None)

    return gather_kernel


_gather = _make_gather()


def kernel(x, weight):
    idx = x.reshape(B_TOTAL // GROUP, GROUP).astype(jnp.int32)
    out = _gather(weight, idx)
    return out.reshape(x.shape[0], x.shape[1], HIDDEN)


# speedup vs baseline: 1.0336x; 1.0336x over previous
import functools

import jax
import jax.numpy as jnp
from jax import lax
from jax.experimental import pallas as pl
from jax.experimental.pallas import tpu as pltpu
from jax.experimental.pallas import tpu_sc as plsc

VOCAB = 1_000_000
HIDDEN = 64
B_TOTAL = 16384 * 200
NC, NS = 2, 16
NW = NC * NS
GROUP = 128
K = 5
CHUNK = K * GROUP
NBUF = 3
ROWS_PER_W = B_TOTAL // NW
STEPS = ROWS_PER_W // CHUNK


def _make_gather():
    mesh = plsc.VectorSubcoreMesh(core_axis_name="c", subcore_axis_name="s")

    @functools.partial(
        pl.kernel,
        mesh=mesh,
        out_type=jax.ShapeDtypeStruct((B_TOTAL, HIDDEN), jnp.float32),
        scratch_types=[
            pltpu.VMEM((NBUF, K, GROUP), jnp.int32),
            pltpu.VMEM((NBUF, CHUNK, HIDDEN), jnp.float32),
            pltpu.SemaphoreType.DMA,
            pltpu.SemaphoreType.DMA,
        ],
        compiler_params=pltpu.CompilerParams(use_tc_tiling_on_sc=False),
    )
    def gather_kernel(table_hbm, idx_hbm, out_hbm, idx_v, rows_v, semg, semo):
        wid = lax.axis_index("s") * NC + lax.axis_index("c")
        base_group = wid * (ROWS_PER_W // GROUP)

        def fire(c):
            b = lax.rem(c, NBUF)
            g0 = base_group + c * K
            pltpu.sync_copy(idx_hbm.at[pl.ds(g0, K)], idx_v.at[b])
            for j in range(K):
                pltpu.async_copy(
                    table_hbm.at[idx_v.at[b].at[j]],
                    rows_v.at[b].at[pl.ds(j * GROUP, GROUP)],
                    semg,
                )

        def wait_gathers(b):
            pltpu.make_async_copy(
                table_hbm.at[pl.ds(0, CHUNK)], rows_v.at[b], semg
            ).wait()

        def fire_out(c):
            b = lax.rem(c, NBUF)
            g0 = base_group + c * K
            pltpu.async_copy(
                rows_v.at[b], out_hbm.at[pl.ds(g0 * GROUP, CHUNK)], semo
            )

        def wait_out():
            pltpu.make_async_copy(
                rows_v.at[0], out_hbm.at[pl.ds(base_group * GROUP, CHUNK)], semo
            ).wait()

        fire(0)
        fire(1)
        wait_gathers(0)
        fire_out(0)
        fire(2)

        def step(c, _):
            wait_out()
            fire(c)
            wait_gathers(lax.rem(c - 2, NBUF))
            fire_out(c - 2)
            return _

        lax.fori_loop(3, STEPS, step, None)

        for c in (STEPS - 2, STEPS - 1):
            wait_gathers(c % NBUF)
            fire_out(c)
        for _ in range(3):
            wait_out()

    return gather_kernel


_gather = _make_gather()


def kernel(x, weight):
    idx = x.reshape(B_TOTAL // GROUP, GROUP).astype(jnp.int32)
    out = _gather(weight, idx)
    return out.reshape(x.shape[0], x.shape[1], HIDDEN)


# speedup vs baseline: 1.0353x; 1.0017x over previous
import functools

import jax
import jax.numpy as jnp
from jax import lax
from jax.experimental import pallas as pl
from jax.experimental.pallas import tpu as pltpu
from jax.experimental.pallas import tpu_sc as plsc

VOCAB = 1_000_000
HIDDEN = 64
SEQS = 16384
SEQLEN = 200
NC, NS = 2, 16
NW = NC * NS
SEQ_PER_W = SEQS // NW
S = 2
NBUF = 4
STEPS = SEQ_PER_W // S
BLK = 32
CPB = BLK // S
SPLIT = (0, 128), (128, SEQLEN - 128)


def _make_gather():
    mesh = plsc.VectorSubcoreMesh(core_axis_name="c", subcore_axis_name="s")

    @functools.partial(
        pl.kernel,
        mesh=mesh,
        out_type=jax.ShapeDtypeStruct((SEQS, SEQLEN, HIDDEN), jnp.float32),
        scratch_types=[
            pltpu.VMEM((2, BLK, SEQLEN), jnp.int32),
            pltpu.VMEM((NBUF, S, SEQLEN, HIDDEN), jnp.float32),
            pltpu.SemaphoreType.DMA,
            pltpu.SemaphoreType.DMA,
        ],
        compiler_params=pltpu.CompilerParams(use_tc_tiling_on_sc=False),
    )
    def gather_kernel(table_hbm, idx_hbm, out_hbm, idx_v, rows_v, semg, semo):
        wid = lax.axis_index("s") * NC + lax.axis_index("c")
        seq0 = wid * SEQ_PER_W

        def load_blk(bk):
            pltpu.sync_copy(
                idx_hbm.at[pl.ds(seq0 + bk * BLK, BLK)],
                idx_v.at[lax.rem(bk, 2)],
            )

        def fire(c):
            b = lax.rem(c, NBUF)
            q = lax.rem(lax.div(c, CPB), 2)
            r0 = lax.rem(c, CPB) * S
            for s in range(S):
                for off, ln in SPLIT:
                    pltpu.async_copy(
                        table_hbm.at[idx_v.at[q, r0 + s, pl.ds(off, ln)]],
                        rows_v.at[b, s, pl.ds(off, ln)],
                        semg,
                    )

        def wait_gathers(b):
            pltpu.make_async_copy(
                out_hbm.at[pl.ds(0, S)], rows_v.at[b], semg
            ).wait()

        def fire_out(c):
            b = lax.rem(c, NBUF)
            pltpu.async_copy(
                rows_v.at[b], out_hbm.at[pl.ds(seq0 + c * S, S)], semo
            )

        def wait_out():
            pltpu.make_async_copy(
                rows_v.at[0], out_hbm.at[pl.ds(seq0, S)], semo
            ).wait()

        load_blk(0)
        fire(0)
        fire(1)
        fire(2)
        wait_gathers(0)
        fire_out(0)
        fire(3)
        wait_gathers(1)
        fire_out(1)

        def step(c, _):
            wait_out()
            @pl.when(lax.rem(c, CPB) == 0)
            def _():
                load_blk(lax.div(c, CPB))
            fire(c)
            wait_gathers(lax.rem(c - 2, NBUF))
            fire_out(c - 2)
            return _

        lax.fori_loop(4, STEPS, step, None)

        for c in (STEPS - 2, STEPS - 1):
            wait_gathers(c % NBUF)
            fire_out(c)
        for _ in range(4):
            wait_out()

    return gather_kernel


_gather = _make_gather()


def kernel(x, weight):
    return _gather(weight, x.astype(jnp.int32))
